# baseline (speedup 1.0000x reference)
import functools

import jax
import jax.numpy as jnp
from jax import lax
from jax.experimental import pallas as pl
from jax.experimental.pallas import tpu as pltpu
from jax.experimental.pallas import tpu_sc as plsc

NC, NS, L = 2, 16, 16
NW = NC * NS
B, S, D = 4, 4096, 1024
SCALE = 32.0
PW = S // NW
CP = 16
NCHUNK = PW // CP
NJ = NCHUNK * B
NB = 3
AH = 1
NSLOT = 2
GROUPS = D // L

_mesh = plsc.VectorSubcoreMesh(core_axis_name="c", subcore_axis_name="s")


@functools.partial(
    pl.kernel,
    out_type=jax.ShapeDtypeStruct((B, S, D), jnp.float32),
    mesh=_mesh,
    scratch_types=[
        pltpu.VMEM((B, PW), jnp.int32),
        pltpu.VMEM((CP, D), jnp.float32),
        pltpu.VMEM((CP, D), jnp.float32),
        *[pltpu.VMEM((CP, D), jnp.float32) for _ in range(NB)],
        pltpu.VMEM_SHARED((NS, NSLOT, CP, D), jnp.float32),
        pltpu.SemaphoreType.DMA,
        pltpu.SemaphoreType.DMA,
        pltpu.SemaphoreType.DMA,
        *[pltpu.SemaphoreType.DMA for _ in range(NB)],
        *[pltpu.SemaphoreType.DMA for _ in range(NB)],
        *[pltpu.SemaphoreType.DMA for _ in range(NSLOT)],
    ],
)
def _emb_kernel(inputs_hbm, tok_hbm, pos_hbm, out_hbm, idx_v, pos0, pos1,
                *rest):
    tok = list(rest[:NB])
    spm = rest[NB]
    si = rest[NB + 1]
    sp = [rest[NB + 2], rest[NB + 3]]
    sg = list(rest[NB + 4:NB + 4 + NB])
    so = list(rest[NB + 4 + NB:NB + 4 + 2 * NB])
    sd = list(rest[NB + 4 + 2 * NB:NB + 4 + 2 * NB + NSLOT])
    posb = [pos0, pos1]

    sid = lax.axis_index("s")
    wid = sid * NC + lax.axis_index("c")
    p0 = wid * PW

    hidx = [
        pltpu.make_async_copy(inputs_hbm.at[b, pl.ds(p0, PW)], idx_v.at[b], si)
        for b in range(B)
    ]
    for h in hidx:
        h.start()
    for h in hidx:
        h.wait()

    def start_pos(c):
        h = pltpu.make_async_copy(
            pos_hbm.at[pl.ds(p0 + c * CP, CP)], posb[c % 2], sp[c % 2])
        h.start()
        return h

    def start_gather(j):
        c, b = j // B, j % B
        nb = j % NB
        h = pltpu.make_async_copy(
            tok_hbm.at[idx_v.at[b, pl.ds(c * CP, CP)]], tok[nb], sg[nb])
        h.start()
        return h

    def start_stage(j):
        nb = j % NB
        h = pltpu.make_async_copy(tok[nb], spm.at[sid, j % NSLOT], so[nb])
        h.start()
        return h

    def start_drain(j):
        c, b = j // B, j % B
        h = pltpu.make_async_copy(
            spm.at[sid, j % NSLOT],
            out_hbm.at[b, pl.ds(p0 + c * CP, CP)], sd[j % NSLOT])
        h.start()
        return h

    hp = [start_pos(0), start_pos(1)]
    hg = [None] * NB
    ho = [None] * NB
    hd = [None] * NSLOT
    for k in range(AH):
        hg[k % NB] = start_gather(k)

    for j in range(NJ):
        c, b = j // B, j % B
        nb = j % NB
        jn = j + AH
        if jn < NJ:
            cn, bn = jn // B, jn % B
            tb = jn % NB
            if bn == 0 and cn >= 2:
                hp[cn % 2] = start_pos(cn)
            hg[tb] = start_gather(jn)

        hg[nb].wait()
        if b == 0:
            hp[c % 2].wait()

        tbuf = tok[nb]
        pbuf = posb[c % 2]

        @pl.loop(0, CP * GROUPS, unroll=8)
        def _fma(g):
            r = g // GROUPS
            off = (g % GROUPS) * L
            t = tbuf[r, pl.ds(off, L)]
            p = pbuf[r, pl.ds(off, L)]
            tbuf[r, pl.ds(off, L)] = t * SCALE + p

        if hd[j % NSLOT] is not None:
            hd[j % NSLOT].wait()
        ho[nb] = start_stage(j)
        if j >= 1:
            ho[(j - 1) % NB].wait()
            hd[(j - 1) % NSLOT] = start_drain(j - 1)

    ho[(NJ - 1) % NB].wait()
    hd[(NJ - 1) % NSLOT] = start_drain(NJ - 1)
    for k in range(NSLOT):
        if hd[k] is not None:
            hd[k].wait()


def kernel(inputs, tok_table, pos_table):
    return _emb_kernel(inputs.astype(jnp.int32), tok_table, pos_table)
